# baseline (speedup 1.0000x reference)
import functools

import jax
import jax.numpy as jnp
from jax import lax
from jax.experimental import pallas as pl
from jax.experimental.pallas import tpu as pltpu
from jax.experimental.pallas import tpu_sc as plsc

B = 16384
C = 100
S_SCALE = 30.0
NC = 2
NS = 16
L = 16
NW = NC * NS
RPW = B // NW
NGROUP = RPW // L
NACC = 8

_LN2 = 0.6931471805599453
_SQRT2 = 1.4142135623730951

_GDN = lax.GatherDimensionNumbers(
    offset_dims=(), collapsed_slice_dims=(0,), start_index_map=(0,))


def _vgather16(vec, idx):
    return lax.gather(vec, idx[:, None], dimension_numbers=_GDN,
                      slice_sizes=(1,),
                      mode=lax.GatherScatterMode.PROMISE_IN_BOUNDS)


def _ldam_body(xf_hbm, t_hbm, m_hbm, out_hbm,
               slab, tv, mvv, accv, sem_slab):
    wid = lax.axis_index("s") * NC + lax.axis_index("c")
    base = wid * RPW
    col_copies = [
        pltpu.async_copy(
            xf_hbm.at[pl.ds(c * B + base, RPW)],
            slab.at[pl.ds(c * RPW, RPW)], sem_slab)
        for c in range(C)
    ]
    pltpu.sync_copy(t_hbm.at[pl.ds(base, RPW)], tv)
    pltpu.sync_copy(m_hbm, mvv.at[pl.ds(0, C)])

    zero = jnp.zeros((L,), jnp.float32)
    iot = lax.iota(jnp.int32, L)
    lane_masks = [iot == i for i in range(L)]
    mreg = [mvv[pl.ds(16 * k, 16)] for k in range(7)]

    with jax.named_scope("slab_wait"):
        for cp in col_copies:
            cp.wait()

    def margin_body(j, zt):
        r0 = j * L
        tt = tv[pl.ds(r0, L)]
        low = lax.bitwise_and(tt, 15)
        hi = lax.shift_right_logical(tt, 4)
        mt = _vgather16(mreg[0], low)
        for k in range(1, 7):
            mt = jnp.where(hi == k, _vgather16(mreg[k], low), mt)
        for i in range(L):
            off = tt[i] * RPW + r0
            v = slab[pl.ds(off, L)]
            v2 = jnp.where(lane_masks[i], v - mt, v)
            zt = zt + jnp.where(lane_masks[i], v2, 0.0)
            slab[pl.ds(off, L)] = v2
        return zt

    with jax.named_scope("margin_rmw"):
        ztacc = lax.fori_loop(0, NGROUP, margin_body, zero)

    def group_body(g, acc):
        r0 = g * L
        mxs = [slab[pl.ds(c * RPW + r0, L)] for c in range(NACC)]
        for c in range(NACC, C):
            mxs[c % NACC] = jnp.maximum(mxs[c % NACC],
                                        slab[pl.ds(c * RPW + r0, L)])
        mx = mxs[0]
        for a in range(1, NACC):
            mx = jnp.maximum(mx, mxs[a])
        kk = S_SCALE * mx
        sss = [zero] * NACC
        for c in range(C):
            v = slab[pl.ds(c * RPW + r0, L)]
            sss[c % NACC] = sss[c % NACC] + jnp.exp(S_SCALE * v - kk)
        ss = sss[0]
        for a in range(1, NACC):
            ss = ss + sss[a]
        bits = lax.bitcast_convert_type(ss, jnp.int32)
        ex = lax.shift_right_arithmetic(bits, 23) - 127
        mf = lax.bitcast_convert_type(
            lax.bitwise_or(lax.bitwise_and(bits, 0x7FFFFF), 0x3F800000),
            jnp.float32)
        big = mf > _SQRT2
        mf = jnp.where(big, mf * 0.5, mf)
        ex = jnp.where(big, ex + 1, ex)
        u = (mf - 1.0) / (mf + 1.0)
        u2 = u * u
        ln = ex.astype(jnp.float32) * _LN2 + 2.0 * u * (
            1.0 + u2 * (0.3333333333 + u2 * 0.2))
        return acc + (kk + ln)

    with jax.named_scope("dense"):
        acc = lax.fori_loop(0, NGROUP, group_body, zero)

    accv[...] = (acc - S_SCALE * ztacc) * (1.0 / B)
    pltpu.sync_copy(accv, out_hbm.at[wid])


_ldam_sc = functools.partial(
    pl.kernel,
    out_type=jax.ShapeDtypeStruct((NW, L), jnp.float32),
    mesh=plsc.VectorSubcoreMesh(core_axis_name="c", subcore_axis_name="s"),
    compiler_params=pltpu.CompilerParams(use_tc_tiling_on_sc=True),
    scratch_types=[
        pltpu.VMEM((C * RPW,), jnp.float32),
        pltpu.VMEM((RPW,), jnp.int32),
        pltpu.VMEM((112,), jnp.float32),
        pltpu.VMEM((L,), jnp.float32),
        pltpu.SemaphoreType.DMA,
    ],
)(_ldam_body)


def _sum_body(p_ref, o_ref):
    o_ref[0, 0] = jnp.sum(p_ref[...])


_sum_tc = pl.pallas_call(
    _sum_body,
    out_shape=jax.ShapeDtypeStruct((1, 1), jnp.float32),
    out_specs=pl.BlockSpec(memory_space=pltpu.SMEM),
)


def kernel(x, target, m_list):
    parts = _ldam_sc(x.T.reshape(-1), target, m_list)
    return _sum_tc(parts)[0, 0]


# speedup vs baseline: 1.1138x; 1.1138x over previous
import functools

import jax
import jax.numpy as jnp
from jax import lax
from jax.experimental import pallas as pl
from jax.experimental.pallas import tpu as pltpu
from jax.experimental.pallas import tpu_sc as plsc

B = 16384
C = 100
S_SCALE = 30.0
NC = 2
NS = 16
L = 16
NW = NC * NS
RPW = B // NW
NGROUP = RPW // L
NACC = 8

_LN2 = 0.6931471805599453
_SQRT2 = 1.4142135623730951

_GDN = lax.GatherDimensionNumbers(
    offset_dims=(), collapsed_slice_dims=(0,), start_index_map=(0,))


def _vgather16(vec, idx):
    return lax.gather(vec, idx[:, None], dimension_numbers=_GDN,
                      slice_sizes=(1,),
                      mode=lax.GatherScatterMode.PROMISE_IN_BOUNDS)


def _ldam_body(xf_hbm, t_hbm, m_hbm, out_hbm,
               slab, tv, mvv, accv, sem_slab):
    wid = lax.axis_index("s") * NC + lax.axis_index("c")
    base = wid * RPW
    col_copies = [
        pltpu.async_copy(
            xf_hbm.at[c, pl.ds(base, RPW)],
            slab.at[pl.ds(c * RPW, RPW)], sem_slab)
        for c in range(C)
    ]
    pltpu.sync_copy(t_hbm.at[pl.ds(base, RPW)], tv)
    pltpu.sync_copy(m_hbm, mvv.at[pl.ds(0, C)])

    zero = jnp.zeros((L,), jnp.float32)
    iot = lax.iota(jnp.int32, L)
    lane_masks = [iot == i for i in range(L)]
    mreg = [mvv[pl.ds(16 * k, 16)] for k in range(7)]

    with jax.named_scope("slab_wait"):
        for cp in col_copies:
            cp.wait()

    def margin_body(j, zt):
        r0 = j * L
        tt = tv[pl.ds(r0, L)]
        low = lax.bitwise_and(tt, 15)
        hi = lax.shift_right_logical(tt, 4)
        mt = _vgather16(mreg[0], low)
        for k in range(1, 7):
            mt = jnp.where(hi == k, _vgather16(mreg[k], low), mt)
        for i in range(L):
            off = tt[i] * RPW + r0
            v = slab[pl.ds(off, L)]
            v2 = jnp.where(lane_masks[i], v - mt, v)
            zt = zt + jnp.where(lane_masks[i], v2, 0.0)
            slab[pl.ds(off, L)] = v2
        return zt

    with jax.named_scope("margin_rmw"):
        ztacc = lax.fori_loop(0, NGROUP, margin_body, zero)

    def group_body(g, acc):
        r0 = g * L
        mxs = [slab[pl.ds(c * RPW + r0, L)] for c in range(NACC)]
        for c in range(NACC, C):
            mxs[c % NACC] = jnp.maximum(mxs[c % NACC],
                                        slab[pl.ds(c * RPW + r0, L)])
        mx = mxs[0]
        for a in range(1, NACC):
            mx = jnp.maximum(mx, mxs[a])
        kk = S_SCALE * mx
        sss = [zero] * NACC
        for c in range(C):
            v = slab[pl.ds(c * RPW + r0, L)]
            sss[c % NACC] = sss[c % NACC] + jnp.exp(S_SCALE * v - kk)
        ss = sss[0]
        for a in range(1, NACC):
            ss = ss + sss[a]
        bits = lax.bitcast_convert_type(ss, jnp.int32)
        ex = lax.shift_right_arithmetic(bits, 23) - 127
        mf = lax.bitcast_convert_type(
            lax.bitwise_or(lax.bitwise_and(bits, 0x7FFFFF), 0x3F800000),
            jnp.float32)
        big = mf > _SQRT2
        mf = jnp.where(big, mf * 0.5, mf)
        ex = jnp.where(big, ex + 1, ex)
        u = (mf - 1.0) / (mf + 1.0)
        u2 = u * u
        ln = ex.astype(jnp.float32) * _LN2 + 2.0 * u * (
            1.0 + u2 * (0.3333333333 + u2 * 0.2))
        return acc + (kk + ln)

    with jax.named_scope("dense"):
        acc = lax.fori_loop(0, NGROUP, group_body, zero)

    accv[...] = (acc - S_SCALE * ztacc) * (1.0 / B)
    pltpu.sync_copy(accv, out_hbm.at[wid])


_ldam_sc = functools.partial(
    pl.kernel,
    out_type=jax.ShapeDtypeStruct((NW, L), jnp.float32),
    mesh=plsc.VectorSubcoreMesh(core_axis_name="c", subcore_axis_name="s"),
    compiler_params=pltpu.CompilerParams(use_tc_tiling_on_sc=True),
    scratch_types=[
        pltpu.VMEM((C * RPW,), jnp.float32),
        pltpu.VMEM((RPW,), jnp.int32),
        pltpu.VMEM((112,), jnp.float32),
        pltpu.VMEM((L,), jnp.float32),
        pltpu.SemaphoreType.DMA,
    ],
)(_ldam_body)


def _sum_body(p_ref, o_ref):
    o_ref[0, 0] = jnp.sum(p_ref[...])


_sum_tc = pl.pallas_call(
    _sum_body,
    out_shape=jax.ShapeDtypeStruct((1, 1), jnp.float32),
    out_specs=pl.BlockSpec(memory_space=pltpu.SMEM),
)


def kernel(x, target, m_list):
    parts = _ldam_sc(x.T, target, m_list)
    return _sum_tc(parts)[0, 0]


# speedup vs baseline: 1.1481x; 1.0308x over previous
import functools

import jax
import jax.numpy as jnp
from jax import lax
from jax.experimental import pallas as pl
from jax.experimental.pallas import tpu as pltpu
from jax.experimental.pallas import tpu_sc as plsc

B = 16384
C = 100
S_SCALE = 30.0
NC = 2
NS = 16
L = 16
NW = NC * NS
RPW = B // NW
NGROUP = RPW // L
NACC = 8

_LN2 = 0.6931471805599453
_SQRT2 = 1.4142135623730951

_GDN = lax.GatherDimensionNumbers(
    offset_dims=(), collapsed_slice_dims=(0,), start_index_map=(0,))


def _vgather16(vec, idx):
    return lax.gather(vec, idx[:, None], dimension_numbers=_GDN,
                      slice_sizes=(1,),
                      mode=lax.GatherScatterMode.PROMISE_IN_BOUNDS)


def _ldam_body(xf_hbm, t_hbm, m_hbm, out_hbm,
               slab, tv, mvv, accv, sem_slab):
    wid = lax.axis_index("s") * NC + lax.axis_index("c")
    base = wid * RPW
    slab_copy = pltpu.async_copy(
        xf_hbm.at[:, pl.ds(base, RPW)], slab, sem_slab)
    pltpu.sync_copy(t_hbm.at[pl.ds(base, RPW)], tv)
    pltpu.sync_copy(m_hbm, mvv.at[pl.ds(0, C)])

    zero = jnp.zeros((L,), jnp.float32)
    iot = lax.iota(jnp.int32, L)
    lane_masks = [iot == i for i in range(L)]
    mreg = [mvv[pl.ds(16 * k, 16)] for k in range(7)]

    slab_copy.wait()

    def margin_body(j, zt):
        r0 = j * L
        tt = tv[pl.ds(r0, L)]
        low = lax.bitwise_and(tt, 15)
        hi = lax.shift_right_logical(tt, 4)
        mt = _vgather16(mreg[0], low)
        for k in range(1, 7):
            mt = jnp.where(hi == k, _vgather16(mreg[k], low), mt)
        for i in range(L):
            ti = tt[i]
            v = slab[ti, pl.ds(r0, L)]
            v2 = jnp.where(lane_masks[i], v - mt, v)
            zt = zt + jnp.where(lane_masks[i], v2, 0.0)
            slab[ti, pl.ds(r0, L)] = v2
        return zt

    ztacc = lax.fori_loop(0, NGROUP, margin_body, zero)

    def group_body(g, acc):
        r0 = g * L
        mxs = [slab[c, pl.ds(r0, L)] for c in range(NACC)]
        for c in range(NACC, C):
            mxs[c % NACC] = jnp.maximum(mxs[c % NACC],
                                        slab[c, pl.ds(r0, L)])
        mx = mxs[0]
        for a in range(1, NACC):
            mx = jnp.maximum(mx, mxs[a])
        kk = S_SCALE * mx
        sss = [zero] * NACC
        for c in range(C):
            v = slab[c, pl.ds(r0, L)]
            sss[c % NACC] = sss[c % NACC] + jnp.exp(S_SCALE * v - kk)
        ss = sss[0]
        for a in range(1, NACC):
            ss = ss + sss[a]
        bits = lax.bitcast_convert_type(ss, jnp.int32)
        ex = lax.shift_right_arithmetic(bits, 23) - 127
        mf = lax.bitcast_convert_type(
            lax.bitwise_or(lax.bitwise_and(bits, 0x7FFFFF), 0x3F800000),
            jnp.float32)
        big = mf > _SQRT2
        mf = jnp.where(big, mf * 0.5, mf)
        ex = jnp.where(big, ex + 1, ex)
        u = (mf - 1.0) / (mf + 1.0)
        u2 = u * u
        ln = ex.astype(jnp.float32) * _LN2 + 2.0 * u * (
            1.0 + u2 * (0.3333333333 + u2 * 0.2))
        return acc + (kk + ln)

    acc = lax.fori_loop(0, NGROUP, group_body, zero)

    accv[...] = (acc - S_SCALE * ztacc) * (1.0 / B)
    pltpu.sync_copy(accv, out_hbm.at[wid])


_ldam_sc = functools.partial(
    pl.kernel,
    out_type=jax.ShapeDtypeStruct((NW, L), jnp.float32),
    mesh=plsc.VectorSubcoreMesh(core_axis_name="c", subcore_axis_name="s"),
    compiler_params=pltpu.CompilerParams(use_tc_tiling_on_sc=True),
    scratch_types=[
        pltpu.VMEM((C, RPW), jnp.float32),
        pltpu.VMEM((RPW,), jnp.int32),
        pltpu.VMEM((112,), jnp.float32),
        pltpu.VMEM((L,), jnp.float32),
        pltpu.SemaphoreType.DMA,
    ],
)(_ldam_body)


def _sum_body(p_ref, o_ref):
    o_ref[0, 0] = jnp.sum(p_ref[...])


_sum_tc = pl.pallas_call(
    _sum_body,
    out_shape=jax.ShapeDtypeStruct((1, 1), jnp.float32),
    out_specs=pl.BlockSpec(memory_space=pltpu.SMEM),
)


def kernel(x, target, m_list):
    parts = _ldam_sc(x.T, target, m_list)
    return _sum_tc(parts)[0, 0]
